# Initial kernel scaffold; baseline (speedup 1.0000x reference)
import jax
import jax.numpy as jnp
from jax.experimental import pallas as pl


def kernel(pcl, params):
    raise NotImplementedError("write your pallas kernel here")


# speedup vs baseline: 12.1724x; 12.1724x over previous
import functools

import jax
import jax.numpy as jnp
import numpy as np
from jax import lax
from jax.experimental import pallas as pl
from jax.experimental.pallas import tpu as pltpu

EPSBN = 1e-5
_MXU_DT = jnp.bfloat16


def _fps_body(xyz_ref, idx_ref, nxyz_ref, *, npoint, rows):
    x = xyz_ref[0, 0]
    y = xyz_ref[0, 1]
    z = xyz_ref[0, 2]
    ii = (lax.broadcasted_iota(jnp.int32, (rows, 128), 0) * 128
          + lax.broadcasted_iota(jnp.int32, (rows, 128), 1)
          ).astype(jnp.float32)
    nf = jnp.float32(rows * 128)

    def step(t, carry):
        dist, far = carry
        idx_ref[0, 0, t] = far.astype(jnp.int32)
        sel = ii == far
        cx = jnp.sum(jnp.where(sel, x, 0.0))
        cy = jnp.sum(jnp.where(sel, y, 0.0))
        cz = jnp.sum(jnp.where(sel, z, 0.0))
        nxyz_ref[0, 0, t] = cx
        nxyz_ref[0, 1, t] = cy
        nxyz_ref[0, 2, t] = cz
        dx = x - cx
        dy = y - cy
        dz = z - cz
        d = (dx * dx + dy * dy) + dz * dz
        dist = jnp.minimum(dist, d)
        mv = jnp.max(dist)
        far_new = jnp.min(jnp.where(dist == mv, ii, nf))
        return dist, far_new

    init = (jnp.full((rows, 128), 1e10, jnp.float32), jnp.float32(0.0))
    lax.fori_loop(0, npoint, step, init)


def _fps(xyz_t, npoint):
    b, _, n = xyz_t.shape
    rows = n // 128
    xyz4 = xyz_t.reshape(b, 3, rows, 128)
    kern = functools.partial(_fps_body, npoint=npoint, rows=rows)
    return pl.pallas_call(
        kern,
        grid=(b,),
        in_specs=[pl.BlockSpec((1, 3, rows, 128), lambda i: (i, 0, 0, 0))],
        out_specs=[
            pl.BlockSpec((1, 1, npoint), lambda i: (i, 0, 0),
                         memory_space=pltpu.SMEM),
            pl.BlockSpec((1, 3, npoint), lambda i: (i, 0, 0),
                         memory_space=pltpu.SMEM),
        ],
        out_shape=[
            jax.ShapeDtypeStruct((b, 1, npoint), jnp.int32),
            jax.ShapeDtypeStruct((b, 3, npoint), jnp.float32),
        ],
    )(xyz4)


def _sqdist(c, xt):
    cx, cy, cz = c[:, 0:1], c[:, 1:2], c[:, 2:3]
    x, y, z = xt[0:1], xt[1:2], xt[2:3]
    bf = _MXU_DT
    f32 = jnp.float32
    cxb = cx.astype(bf).astype(f32)
    cyb = cy.astype(bf).astype(f32)
    czb = cz.astype(bf).astype(f32)
    xb = x.astype(bf).astype(f32)
    yb = y.astype(bf).astype(f32)
    zb = z.astype(bf).astype(f32)
    dot = (cxb * xb + cyb * yb) + czb * zb
    cn = (cx * cx + cy * cy) + cz * cz
    pn = (x * x + y * y) + z * z
    return (-2.0 * dot + cn) + pn


def _ballq_body(xyzt_ref, nxyz_ref, idx_ref, *, r2, n, k):
    bi = pl.program_id(0)
    d = _sqdist(nxyz_ref[0], xyzt_ref[0])
    ii = lax.broadcasted_iota(jnp.int32, d.shape, 1).astype(jnp.float32)
    nf = jnp.float32(n)
    a = jnp.where(d > r2, nf, ii)
    cols = []
    for _ in range(k):
        m = jnp.min(a, axis=1, keepdims=True)
        cols.append(m)
        a = jnp.where(a == m, nf, a)
    out = jnp.concatenate(cols, axis=1)
    out = jnp.where(out == nf, out[:, 0:1], out)
    out = jnp.minimum(out, nf - 1.0)
    idx_ref[0] = out.astype(jnp.int32) + bi * n


def _ball_query(xyz_t, new_xyz, radius, k, ts):
    b, _, n = xyz_t.shape
    s = new_xyz.shape[1]
    kern = functools.partial(_ballq_body, r2=radius ** 2, n=n, k=k)
    return pl.pallas_call(
        kern,
        grid=(b, s // ts),
        in_specs=[
            pl.BlockSpec((1, 3, n), lambda i, j: (i, 0, 0)),
            pl.BlockSpec((1, ts, 3), lambda i, j: (i, j, 0)),
        ],
        out_specs=pl.BlockSpec((1, ts, k), lambda i, j: (i, j, 0)),
        out_shape=jax.ShapeDtypeStruct((b, s, k), jnp.int32),
    )(xyz_t, new_xyz)


def _apply_layer(x, wt_ref, b_ref, sc_ref, be_ref, relu):
    y = jnp.dot(x.astype(_MXU_DT), wt_ref[...],
                preferred_element_type=jnp.float32) + b_ref[...]
    y = y * sc_ref[...] + be_ref[...]
    if relu:
        y = jnp.maximum(y, 0.0)
    return y


def _sa_post_body(g_ref, c_ref, w1_ref, b1_ref, s1_ref, e1_ref,
                  w2_ref, b2_ref, s2_ref, e2_ref, o_ref, *, k):
    c = c_ref[...]
    acc = None
    for j in range(k):
        x = g_ref[j] - c
        x = _apply_layer(x, w1_ref, b1_ref, s1_ref, e1_ref, True)
        x = _apply_layer(x, w2_ref, b2_ref, s2_ref, e2_ref, True)
        acc = x if acc is None else jnp.maximum(acc, x)
    o_ref[...] = acc


def _sa_post(g, cext, l1, l2, k, ts):
    m, dd = cext.shape
    cout = l2[0].shape[1]
    kern = functools.partial(_sa_post_body, k=k)
    specs = [
        pl.BlockSpec((k, ts, dd), lambda i: (0, i, 0)),
        pl.BlockSpec((ts, dd), lambda i: (i, 0)),
    ]
    args = [g, cext]
    for (wt, b, sc, be) in (l1, l2):
        ci, co = wt.shape
        specs.append(pl.BlockSpec((ci, co), lambda i: (0, 0)))
        specs.append(pl.BlockSpec((1, co), lambda i: (0, 0)))
        specs.append(pl.BlockSpec((1, co), lambda i: (0, 0)))
        specs.append(pl.BlockSpec((1, co), lambda i: (0, 0)))
        args.extend([wt, b, sc, be])
    return pl.pallas_call(
        kern,
        grid=(m // ts,),
        in_specs=specs,
        out_specs=pl.BlockSpec((ts, cout), lambda i: (i, 0)),
        out_shape=jax.ShapeDtypeStruct((m, cout), jnp.float32),
    )(*args)


def _knn3_body(q_ref, x2t_ref, idx_ref, w_ref, *, n2):
    bi = pl.program_id(0)
    d = _sqdist(q_ref[0], x2t_ref[0])
    ii = lax.broadcasted_iota(jnp.int32, d.shape, 1).astype(jnp.float32)
    nf = jnp.float32(n2)
    big = jnp.float32(1e30)
    ms, idxs = [], []
    for _ in range(3):
        m = jnp.min(d, axis=1, keepdims=True)
        i = jnp.min(jnp.where(d == m, ii, nf), axis=1, keepdims=True)
        ms.append(m)
        idxs.append(i)
        d = jnp.where(ii == i, big, d)
    r0 = 1.0 / (ms[0] + 1e-8)
    r1 = 1.0 / (ms[1] + 1e-8)
    r2 = 1.0 / (ms[2] + 1e-8)
    norm = (r0 + r1) + r2
    idx_ref[0] = jnp.concatenate(idxs, axis=1).astype(jnp.int32) + bi * n2
    w_ref[0] = jnp.concatenate([r0 / norm, r1 / norm, r2 / norm], axis=1)


def _knn3(xyz1, xyz2_t, t1):
    b, n1, _ = xyz1.shape
    n2 = xyz2_t.shape[2]
    kern = functools.partial(_knn3_body, n2=n2)
    return pl.pallas_call(
        kern,
        grid=(b, n1 // t1),
        in_specs=[
            pl.BlockSpec((1, t1, 3), lambda i, j: (i, j, 0)),
            pl.BlockSpec((1, 3, n2), lambda i, j: (i, 0, 0)),
        ],
        out_specs=[
            pl.BlockSpec((1, t1, 3), lambda i, j: (i, j, 0)),
            pl.BlockSpec((1, t1, 3), lambda i, j: (i, j, 0)),
        ],
        out_shape=[
            jax.ShapeDtypeStruct((b, n1, 3), jnp.int32),
            jax.ShapeDtypeStruct((b, n1, 3), jnp.float32),
        ],
    )(xyz1, xyz2_t)


def _fp_post_body(*refs, nlayers, has_p1, relus):
    g_ref, w_ref = refs[0], refs[1]
    pos = 2
    p1_ref = None
    if has_p1:
        p1_ref = refs[pos]
        pos += 1
    layer_refs = refs[pos:pos + 4 * nlayers]
    o_ref = refs[pos + 4 * nlayers]
    w = w_ref[...]
    interp = g_ref[0] * w[:, 0:1]
    interp = interp + g_ref[1] * w[:, 1:2]
    interp = interp + g_ref[2] * w[:, 2:3]
    x = interp
    if has_p1:
        x = jnp.concatenate([p1_ref[...], interp], axis=1)
    for li in range(nlayers):
        r = layer_refs[4 * li:4 * li + 4]
        x = _apply_layer(x, r[0], r[1], r[2], r[3], relus[li])
    o_ref[...] = x


def _fp_post(g, w, p1, layers, relus, tn):
    m, c2 = g.shape[1], g.shape[2]
    nlayers = len(layers)
    cout = layers[-1][0].shape[1]
    has_p1 = p1 is not None
    kern = functools.partial(_fp_post_body, nlayers=nlayers, has_p1=has_p1,
                             relus=tuple(relus))
    in_specs = [
        pl.BlockSpec((3, tn, c2), lambda i: (0, i, 0)),
        pl.BlockSpec((tn, 3), lambda i: (i, 0)),
    ]
    args = [g, w]
    if has_p1:
        c1 = p1.shape[1]
        in_specs.append(pl.BlockSpec((tn, c1), lambda i: (i, 0)))
        args.append(p1)
    for (wt, bias, sc, be) in layers:
        ci, co = wt.shape
        for arr in (wt, bias, sc, be):
            in_specs.append(pl.BlockSpec(arr.shape, lambda i: (0, 0)))
            args.append(arr)
    return pl.pallas_call(
        kern,
        grid=(m // tn,),
        in_specs=in_specs,
        out_specs=pl.BlockSpec((tn, cout), lambda i: (i, 0)),
        out_shape=jax.ShapeDtypeStruct((m, cout), jnp.float32),
    )(*args)


def _gather_rows(table, idx):
    return table[idx]


def _prep(layer, pad_in=0):
    w, b, g, be = layer
    wt = w.T.astype(jnp.bfloat16)
    if pad_in:
        wt = jnp.concatenate(
            [wt, jnp.zeros((pad_in, wt.shape[1]), jnp.bfloat16)], axis=0)
    inv = 1.0 / np.sqrt(1.0 + EPSBN)
    return wt, b[None, :], (g * inv)[None, :], be[None, :]


def _set_abstraction(xyz_t, pts, npoint, radius, nsample, layers):
    b, _, n = xyz_t.shape
    d0 = 3 + pts.shape[2]
    dp = -d0 % 16
    l1 = _prep(layers[0], pad_in=dp)
    l2 = _prep(layers[1])

    _, new_xyz_t = _fps(xyz_t, npoint)
    new_xyz = jnp.transpose(new_xyz_t, (0, 2, 1))
    xyz_cl = jnp.transpose(xyz_t, (0, 2, 1))
    feat = jnp.concatenate(
        [xyz_cl, pts, jnp.zeros((b, n, dp), jnp.float32)], axis=2)
    cext = jnp.concatenate(
        [new_xyz, jnp.zeros((b, npoint, d0 + dp - 3), jnp.float32)], axis=2)

    ts = min(npoint, 256)
    idx = _ball_query(xyz_t, new_xyz, radius, nsample, ts)

    idx_km = jnp.transpose(idx.reshape(b * npoint, nsample))
    g = _gather_rows(feat.reshape(b * n, -1), idx_km.reshape(-1))
    g = g.reshape(nsample, b * npoint, -1)
    newp = _sa_post(g, cext.reshape(b * npoint, -1), l1, l2, nsample,
                    ts=min(npoint, 256))
    return new_xyz, new_xyz_t, newp


def _feature_propagation(xyz1, xyz2_t, p1_flat, p2_flat, layers,
                         extra_layers=()):
    b, n1, _ = xyz1.shape
    idx3, w3 = _knn3(xyz1, xyz2_t, t1=min(n1, 256))
    idx_km = jnp.transpose(idx3.reshape(b * n1, 3))
    g = _gather_rows(p2_flat, idx_km.reshape(-1))
    g = g.reshape(3, b * n1, -1)
    prepped = [_prep(l) for l in layers]
    relus = [True] * len(prepped)
    for (wt, bias) in extra_layers:
        co = wt.shape[1]
        prepped.append((wt.astype(jnp.bfloat16), bias,
                        jnp.ones((1, co), jnp.float32),
                        jnp.zeros((1, co), jnp.float32)))
        relus.append(False)
    return _fp_post(g, w3.reshape(b * n1, 3), p1_flat, prepped, relus,
                    tn=min(n1 * b, 512))


def kernel(pcl, params):
    b, _, n = pcl.shape
    xyz_t = pcl[:, :3, :]
    pts = jnp.transpose(pcl, (0, 2, 1))

    s1_xyz, s1_xyz_t, s1_pts = _set_abstraction(xyz_t, pts, 4096, 0.1, 32,
                                                params['sa1'])
    s2_xyz, s2_xyz_t, s2_pts = _set_abstraction(s1_xyz_t,
                                                s1_pts.reshape(b, 4096, -1),
                                                1024, 0.2, 32, params['sa2'])
    s3_xyz, s3_xyz_t, s3_pts = _set_abstraction(s2_xyz_t,
                                                s2_pts.reshape(b, 1024, -1),
                                                512, 0.4, 32, params['sa3'])

    s2_new = _feature_propagation(s2_xyz, s3_xyz_t, s2_pts, s3_pts,
                                  params['fp3'])
    s1_new = _feature_propagation(s1_xyz, s2_xyz_t, s1_pts, s2_new,
                                  params['fp2'])
    wfc, bfc = params['fc']
    fc_layer = (wfc.T, bfc[None, :])
    s0_new = _feature_propagation(jnp.transpose(xyz_t, (0, 2, 1)), s1_xyz_t,
                                  None, s1_new, params['fp1'],
                                  extra_layers=(fc_layer,))
    return s0_new.reshape(b, n, -1)
